# Initial kernel scaffold; baseline (speedup 1.0000x reference)
import jax
import jax.numpy as jnp
from jax.experimental import pallas as pl


def kernel(x, edge_index, adj_values):
    raise NotImplementedError("write your pallas kernel here")


# speedup vs baseline: 3.7696x; 3.7696x over previous
import functools

import jax
import jax.numpy as jnp
from jax import lax
from jax.experimental import pallas as pl
from jax.experimental.pallas import tpu as pltpu
from jax.experimental.pallas import tpu_sc as plsc

N_NODES = 10000
D_FEAT = 128
NC = 2
NS = 16
LANES = 16
E_CHUNK = 128
ROWS_PER_TILE = 624
ROWS_TAIL = N_NODES - NS * ROWS_PER_TILE


def _sc_partials(n_chunks_per_tile):
    e_per_tile = n_chunks_per_tile * E_CHUNK

    mesh = plsc.VectorSubcoreMesh(
        core_axis_name="c", subcore_axis_name="s", num_cores=NC, num_subcores=NS
    )

    @functools.partial(
        pl.kernel,
        out_type=jax.ShapeDtypeStruct((NC, N_NODES, D_FEAT), jnp.float32),
        mesh=mesh,
        scratch_types=[
            pltpu.VMEM((E_CHUNK,), jnp.int32),
            pltpu.VMEM((E_CHUNK,), jnp.int32),
            pltpu.VMEM((E_CHUNK,), jnp.float32),
            pltpu.VMEM((E_CHUNK, D_FEAT), jnp.float32),
            pltpu.VMEM_SHARED((N_NODES, D_FEAT), jnp.float32),
            pltpu.SemaphoreType.DMA,
        ],
    )
    def k(src_hbm, dst_hbm, adj_hbm, x_hbm, out_hbm, idx_s, idx_d, adjv, rows, acc, sem):
        cid = lax.axis_index("c")
        sid = lax.axis_index("s")

        def zero_row(r, _):
            for j in range(D_FEAT // LANES):
                rows[r, pl.ds(j * LANES, LANES)] = jnp.zeros((LANES,), jnp.float32)
            return _

        lax.fori_loop(0, E_CHUNK, zero_row, None)

        row_base = sid * ROWS_PER_TILE
        n_full = ROWS_PER_TILE // E_CHUNK
        rem = ROWS_PER_TILE - n_full * E_CHUNK
        for kk in range(n_full):
            pltpu.sync_copy(rows, acc.at[pl.ds(row_base + kk * E_CHUNK, E_CHUNK)])
        pltpu.sync_copy(
            rows.at[pl.ds(0, rem)],
            acc.at[pl.ds(row_base + n_full * E_CHUNK, rem)],
        )

        @pl.when(sid == NS - 1)
        def _zero_tail():
            pltpu.sync_copy(
                rows.at[pl.ds(0, ROWS_TAIL)],
                acc.at[pl.ds(NS * ROWS_PER_TILE, ROWS_TAIL)],
            )

        plsc.subcore_barrier()

        tile_id = cid * NS + sid
        e_base = tile_id * e_per_tile

        def chunk_body(g, _):
            base = e_base + g * E_CHUNK
            pltpu.sync_copy(src_hbm.at[pl.ds(base, E_CHUNK)], idx_s)
            pltpu.sync_copy(dst_hbm.at[pl.ds(base, E_CHUNK)], idx_d)
            pltpu.sync_copy(adj_hbm.at[pl.ds(base, E_CHUNK)], adjv)
            pltpu.async_copy(x_hbm.at[idx_s], rows, sem).wait()

            def scale_16(t, _2):
                e0 = t * LANES
                a16 = adjv[pl.ds(e0, LANES)]
                for l in range(LANES):
                    a = a16[l]
                    for j in range(D_FEAT // LANES):
                        sl = pl.ds(j * LANES, LANES)
                        rows[e0 + l, sl] = rows[e0 + l, sl] * a
                return _2

            lax.fori_loop(0, E_CHUNK // LANES, scale_16, None)
            pltpu.sync_copy(rows, acc.at[idx_d], add=True)
            return _

        lax.fori_loop(0, n_chunks_per_tile, chunk_body, None)
        plsc.subcore_barrier()

        for kk in range(n_full):
            r0 = row_base + kk * E_CHUNK
            pltpu.sync_copy(acc.at[pl.ds(r0, E_CHUNK)], rows)
            pltpu.sync_copy(rows, out_hbm.at[cid, pl.ds(r0, E_CHUNK)])
        r0 = row_base + n_full * E_CHUNK
        pltpu.sync_copy(acc.at[pl.ds(r0, rem)], rows.at[pl.ds(0, rem)])
        pltpu.sync_copy(rows.at[pl.ds(0, rem)], out_hbm.at[cid, pl.ds(r0, rem)])

        @pl.when(sid == NS - 1)
        def _pub_tail():
            t0 = NS * ROWS_PER_TILE
            pltpu.sync_copy(acc.at[pl.ds(t0, ROWS_TAIL)], rows.at[pl.ds(0, ROWS_TAIL)])
            pltpu.sync_copy(rows.at[pl.ds(0, ROWS_TAIL)], out_hbm.at[cid, pl.ds(t0, ROWS_TAIL)])

    return k


def _combine_body(p_ref, o_ref):
    o_ref[...] = p_ref[0] + p_ref[1]


def _combine(partials):
    rows_blk = 1000
    return pl.pallas_call(
        _combine_body,
        out_shape=jax.ShapeDtypeStruct((N_NODES, D_FEAT), jnp.float32),
        grid=(N_NODES // rows_blk,),
        in_specs=[pl.BlockSpec((NC, rows_blk, D_FEAT), lambda i: (0, i, 0))],
        out_specs=pl.BlockSpec((rows_blk, D_FEAT), lambda i: (i, 0)),
    )(partials)


@jax.jit
def kernel(x, edge_index, adj_values):
    n_edges = edge_index.shape[1]
    per_tile = E_CHUNK * ((n_edges + NC * NS * E_CHUNK - 1) // (NC * NS * E_CHUNK))
    e_pad = per_tile * NC * NS

    dst = edge_index[0].astype(jnp.int32)
    src = edge_index[1].astype(jnp.int32)
    adj = adj_values.astype(jnp.float32)
    pad = e_pad - n_edges
    if pad:
        dst = jnp.concatenate([dst, jnp.zeros((pad,), jnp.int32)])
        src = jnp.concatenate([src, jnp.zeros((pad,), jnp.int32)])
        adj = jnp.concatenate([adj, jnp.zeros((pad,), jnp.float32)])

    partials = _sc_partials(per_tile // E_CHUNK)(src, dst, adj, x)
    return _combine(partials)
